# baseline (speedup 1.0000x reference)
import math

import jax
import jax.numpy as jnp
from jax import lax
from jax.experimental import pallas as pl
from jax.experimental.pallas import tpu as pltpu
from jax.experimental.pallas import tpu_sc as plsc

_N = 10000
_D = 128
_B = 2
_NEG = 4
_E = 160000
_CH = 2048
_NP = 10240
_NT = _B * _NEG
_NSPEC = _B + _NT
_NC = 2
_NSUB = 16
_NW = _NC * _NSUB
_EPW = _E // _NW
_DROWS = _NP // 16
_CSLEN = _NSPEC * _NP + 16


_UNROLL = 8


def _hist_body(ei_hbm, spec_hbm, zcs_hbm, zi_hbm,
               out_deg_hbm, out_cs_hbm,
               e0_v, e1_v, spec_v, hist_v, smark_v, ones_v, shared_cs):
    cid = lax.axis_index("c")
    sid = lax.axis_index("s")
    wid = sid * _NC + cid
    base = wid * _EPW

    @pl.when(sid == 0)
    def _():
        pltpu.sync_copy(zcs_hbm, shared_cs)

    pltpu.sync_copy(ei_hbm.at[pl.ds(base, _EPW)], e0_v.at[pl.ds(0, _EPW)])
    pltpu.sync_copy(ei_hbm.at[pl.ds(_E + base, _EPW)],
                    e1_v.at[pl.ds(0, _EPW)])
    pltpu.sync_copy(spec_hbm, spec_v)
    pltpu.sync_copy(zcs_hbm.at[pl.ds(0, _NP)], hist_v)
    pltpu.sync_copy(zi_hbm, smark_v)
    ones_v[...] = jnp.ones((16,), jnp.float32)

    ones = jnp.ones((16,), jnp.float32)
    lane = lax.iota(jnp.int32, 16)
    bitvals = jnp.where(lane < _NSPEC, jnp.left_shift(1, lane), 0)
    plsc.addupdate_scatter(smark_v, [spec_v[...]], bitvals)
    plsc.subcore_barrier()

    def body(j, carry):
        base_off = j * 16 * _UNROLL
        datas = []
        hitacc = None
        for k in range(_UNROLL):
            start = base_off + k * 16
            valid = lane < (_EPW - start)
            a = e0_v[pl.ds(start, 16)]
            b = e1_v[pl.ds(start, 16)]
            plsc.addupdate_scatter(hist_v, [a], ones, mask=valid)
            plsc.addupdate_scatter(hist_v, [b], ones, mask=valid)
            ma = plsc.load_gather(smark_v, [a], mask=valid)
            mb = plsc.load_gather(smark_v, [b], mask=valid)
            hm = valid & ((ma | mb) != 0)
            hitacc = hm if hitacc is None else (hitacc | hm)
            datas.append((valid, a, b, ma, mb))

        @pl.when(jnp.any(hitacc))
        def _():
            for valid, a, b, ma, mb in datas:
                for s in range(_NSPEC):
                    bit = jnp.int32(1 << s)
                    m0 = valid & ((ma & bit) != 0)

                    @pl.when(jnp.any(m0))
                    def _():
                        idx = jnp.where(m0, s * _NP + b, _NSPEC * _NP + lane)
                        pltpu.sync_copy(ones_v, shared_cs.at[idx], add=True)

                    m1 = valid & ((mb & bit) != 0)

                    @pl.when(jnp.any(m1))
                    def _():
                        idx = jnp.where(m1, s * _NP + a, _NSPEC * _NP + lane)
                        pltpu.sync_copy(ones_v, shared_cs.at[idx], add=True)
        return carry

    lax.fori_loop(0, (_EPW + 16 * _UNROLL - 1) // (16 * _UNROLL), body, 0)

    pltpu.sync_copy(hist_v, out_deg_hbm.at[wid])
    plsc.subcore_barrier()

    @pl.when(sid == 0)
    def _():
        pltpu.sync_copy(shared_cs, out_cs_hbm.at[cid])


def _sc_histograms(ei, spec):
    zcs = jnp.zeros((_CSLEN,), jnp.float32)
    zi = jnp.zeros((_NP,), jnp.int32)
    mesh = plsc.VectorSubcoreMesh(core_axis_name="c", subcore_axis_name="s",
                                  num_cores=_NC, num_subcores=_NSUB)
    epad = _EPW + 16 * _UNROLL
    f = pl.kernel(
        _hist_body,
        out_type=(jax.ShapeDtypeStruct((_NW, _NP), jnp.float32),
                  jax.ShapeDtypeStruct((_NC, _CSLEN), jnp.float32)),
        mesh=mesh,
        compiler_params=pltpu.CompilerParams(needs_layout_passes=False),
        scratch_types=[
            pltpu.VMEM((epad,), jnp.int32),
            pltpu.VMEM((epad,), jnp.int32),
            pltpu.VMEM((16,), jnp.int32),
            pltpu.VMEM((_NP,), jnp.float32),
            pltpu.VMEM((_NP,), jnp.int32),
            pltpu.VMEM((16,), jnp.float32),
            pltpu.VMEM_SHARED((_CSLEN,), jnp.float32),
        ],
    )
    return f(ei, spec, zcs, zi)


_NCH = _NP // _CH


def _c0dot(w_ref, x):
    return jnp.dot(w_ref[...], x, preferred_element_type=jnp.float32)


def _dense_body(degp_ref, csp_ref, hsht_ref, relt_ref, stht_ref,
                w0t_ref, wlht_ref, wlqt_ref, wm1t_ref, wm2t_ref, w1t_ref,
                b0t_ref, b1t_ref, blint_ref, bm1t_ref, bm2_ref,
                h0_ref, t_ref, out_ref, aggsum_ref, aggmax_ref, degt_ref,
                degsum_ref, const_ref, mean_ref):
    i = pl.program_id(0)

    @pl.when(i == 0)
    def _():
        acc = degp_ref[0:1, :]
        for w in range(1, _NW):
            acc = acc + degp_ref[w:w + 1, :]
        degsum_ref[...] = acc
        mean_ref[0, 0] = jnp.sum(jnp.log(acc + 1.0)) / float(_N)

        relt = relt_ref[...]
        hsht = hsht_ref[...]
        gate0 = jax.nn.sigmoid(
            jnp.sum(stht_ref[...] * relt, axis=0, keepdims=True)
            / math.sqrt(float(_D)))
        mt = gate0 * hsht * relt
        const_ref[:, 0:_B] = _c0dot(w0t_ref, mt)
        const_ref[:, _B:2 * _B] = _c0dot(w0t_ref, jnp.maximum(mt, 0.0))
        const_ref[:, 2 * _B:3 * _B] = \
            _c0dot(wlqt_ref, relt) + blint_ref[...]
        const_ref[:, 3 * _B:4 * _B] = hsht * relt

    mean_ld = mean_ref[0, 0]
    off = i * _CH
    deg = degsum_ref[0:1, pl.ds(off, _CH)]
    scal = jnp.log(deg + 1.0) / mean_ld
    degc = jnp.maximum(deg, 1.0)
    lane_ids = lax.broadcasted_iota(jnp.int32, (1, _CH), 1) + off

    def csrow(s):
        return (csp_ref[0:1, pl.ds(s * _NP + off, _CH)]
                + csp_ref[1:2, pl.ds(s * _NP + off, _CH)])

    for bb in range(_B):
        cb = csrow(bb)
        sbt = jnp.concatenate(
            [csrow(_B + bb * _NEG + k) for k in range(_NEG)], axis=0)
        u_b = const_ref[:, bb:bb + 1]
        w_b = const_ref[:, _B + bb:_B + bb + 1]
        q_b = const_ref[:, 2 * _B + bb:2 * _B + bb + 1]
        bnd_b = const_ref[:, 3 * _B + bb:3 * _B + bb + 1]

        a_coef = scal * cb / degc \
            + jnp.where((cb > 0) & (cb == deg), scal, 0.0)
        b_coef = jnp.where((cb > 0) & (cb < deg), scal, 0.0)
        hid1 = jnp.maximum(u_b * a_coef + w_b * b_coef + b0t_ref[...], 0.0)
        hid1 = hid1 + jnp.where(lane_ids == h0_ref[bb], 1.0, 0.0) * bnd_b

        z1 = jnp.maximum(_c0dot(wlht_ref, hid1) + q_b, 0.0)
        z2 = jnp.maximum(_c0dot(wm1t_ref, z1) + bm1t_ref[...], 0.0)
        s1 = _c0dot(wm2t_ref, z2) + bm2_ref[...]
        gate1 = jax.nn.sigmoid(s1)
        g = gate1 * hid1

        part_sum = lax.dot_general(g, sbt, (((1,), (1,)), ((), ())),
                                   preferred_element_type=jnp.float32)
        neg_inf = jnp.float32(-jnp.inf)
        maxes = []
        degts = []
        for k in range(_NEG):
            gm = jnp.where(sbt[k:k + 1, :] > 0.0, g, neg_inf)
            maxes.append(jnp.max(gm, axis=1, keepdims=True))
            degts.append(jnp.sum(sbt[k:k + 1, :], axis=1, keepdims=True))
        part_max = jnp.concatenate(maxes, axis=1)
        part_degt = jnp.concatenate(degts, axis=1)

        lo, hi = bb * _NEG, (bb + 1) * _NEG

        @pl.when(i == 0)
        def _():
            aggsum_ref[:, lo:hi] = part_sum
            aggmax_ref[:, lo:hi] = part_max
            degt_ref[:, lo:hi] = part_degt

        @pl.when(i > 0)
        def _():
            aggsum_ref[:, lo:hi] = aggsum_ref[:, lo:hi] + part_sum
            aggmax_ref[:, lo:hi] = jnp.maximum(aggmax_ref[:, lo:hi], part_max)
            degt_ref[:, lo:hi] = degt_ref[:, lo:hi] + part_degt

    @pl.when(i == _NCH - 1)
    def _():
        degt = degt_ref[...]
        scal_t = jnp.log(degt + 1.0) / mean_ld
        agg2 = (aggsum_ref[...] / jnp.maximum(degt, 1.0)
                + jnp.where(degt > 0, aggmax_ref[...], 0.0)) * scal_t
        hid2 = jnp.maximum(_c0dot(w1t_ref, agg2) + b1t_ref[...], 0.0)

        bnd8 = jnp.concatenate(
            [const_ref[:, 3 * _B + bb:3 * _B + bb + 1]
             for bb in range(_B) for _ in range(_NEG)], axis=1)
        q8 = jnp.concatenate(
            [const_ref[:, 2 * _B + bb:2 * _B + bb + 1]
             for bb in range(_B) for _ in range(_NEG)], axis=1)
        tmatch = jnp.concatenate(
            [jnp.where(t_ref[bb * _NEG + k] == h0_ref[bb],
                       1.0, 0.0).reshape(1, 1)
             for bb in range(_B) for k in range(_NEG)], axis=1)
        hid2 = hid2 + tmatch * bnd8

        z1f = jnp.maximum(_c0dot(wlht_ref, hid2) + q8, 0.0)
        z2f = jnp.maximum(_c0dot(wm1t_ref, z1f) + bm1t_ref[...], 0.0)
        s2 = _c0dot(wm2t_ref, z2f) + bm2_ref[...]
        out_ref[...] = s2


def kernel(h_index, r_index, t_index, hidden_states, rel_hidden_states, x,
           edge_index, score_text_embs, all_index, rel_table, W0, b0, W1, b1,
           W_lin, b_lin, W_mlp1, b_mlp1, W_mlp2, b_mlp2):
    ei = edge_index.astype(jnp.int32).reshape(-1)
    h0 = h_index[:, 0].astype(jnp.int32)
    r0 = r_index[:, 0].astype(jnp.int32)
    t = t_index.astype(jnp.int32).reshape(-1)

    spec = jnp.concatenate(
        [h0, t, _N + jnp.arange(16 - _NSPEC, dtype=jnp.int32)])
    degp, cs_part = _sc_histograms(ei, spec)

    relt = rel_table[r0].T
    hsht = hidden_states[h0].T
    stht = score_text_embs[h0].T

    full = lambda shape: pl.BlockSpec(shape, lambda i: (0,) * len(shape))
    out = pl.pallas_call(
        _dense_body,
        grid=(_NCH,),
        in_specs=[
            full((_NW, _NP)), full((_NC, _CSLEN)),
            full((_D, _B)), full((_D, _B)), full((_D, _B)),
            full((_D, _D)), full((_D, _D)), full((_D, _D)),
            full((2 * _D, _D)), full((1, 2 * _D)), full((_D, _D)),
            full((_D, 1)), full((_D, 1)), full((_D, 1)),
            full((2 * _D, 1)), full((1, 1)),
            pl.BlockSpec(memory_space=pltpu.SMEM),
            pl.BlockSpec(memory_space=pltpu.SMEM),
        ],
        out_specs=pl.BlockSpec((1, _NT), lambda i: (0, 0)),
        out_shape=jax.ShapeDtypeStruct((1, _NT), jnp.float32),
        scratch_shapes=[
            pltpu.VMEM((_D, _NT), jnp.float32),
            pltpu.VMEM((_D, _NT), jnp.float32),
            pltpu.VMEM((1, _NT), jnp.float32),
            pltpu.VMEM((1, _NP), jnp.float32),
            pltpu.VMEM((_D, 4 * _B), jnp.float32),
            pltpu.SMEM((1, 1), jnp.float32),
        ],
    )(degp, cs_part, hsht, relt, stht,
      W0.T, W_lin[:_D].T, W_lin[_D:].T, W_mlp1.T, W_mlp2.T, W1.T,
      b0.reshape(_D, 1), b1.reshape(_D, 1), b_lin.reshape(_D, 1),
      b_mlp1.reshape(2 * _D, 1), b_mlp2.reshape(1, 1), h0, t)
    return out.reshape(_B, _NEG)


# speedup vs baseline: 1.3378x; 1.3378x over previous
import math

import jax
import jax.numpy as jnp
from jax import lax
from jax.experimental import pallas as pl
from jax.experimental.pallas import tpu as pltpu
from jax.experimental.pallas import tpu_sc as plsc

_N = 10000
_D = 128
_B = 2
_NEG = 4
_E = 160000
_CH = 2048
_NP = 10240
_NT = _B * _NEG
_NSPEC = _B + _NT
_NC = 2
_NSUB = 16
_NW = _NC * _NSUB
_EPW = _E // _NW
_DROWS = _NP // 16
_CSLEN = _NSPEC * _NP + 16


_UNROLL = 4


def _hist_body(ei_hbm, spec_hbm, zcs_hbm, zi_hbm,
               out_deg_hbm, out_cs_hbm,
               e0_v, e1_v, spec_v, hist_v, smark_v, ones_v, shared_cs):
    cid = lax.axis_index("c")
    sid = lax.axis_index("s")
    wid = sid * _NC + cid
    base = wid * _EPW

    @pl.when(sid == 0)
    def _():
        pltpu.sync_copy(zcs_hbm, shared_cs)

    pltpu.sync_copy(ei_hbm.at[pl.ds(base, _EPW)], e0_v.at[pl.ds(0, _EPW)])
    pltpu.sync_copy(ei_hbm.at[pl.ds(_E + base, _EPW)],
                    e1_v.at[pl.ds(0, _EPW)])
    pltpu.sync_copy(spec_hbm, spec_v)
    pltpu.sync_copy(zcs_hbm.at[pl.ds(0, _NP)], hist_v)
    pltpu.sync_copy(zi_hbm, smark_v)
    ones_v[...] = jnp.ones((16,), jnp.float32)

    ones = jnp.ones((16,), jnp.float32)
    lane = lax.iota(jnp.int32, 16)
    bitvals = jnp.where(lane < _NSPEC, jnp.left_shift(1, lane), 0)
    plsc.addupdate_scatter(smark_v, [spec_v[...]], bitvals)
    plsc.subcore_barrier()

    def body(j, carry):
        base_off = j * 16 * _UNROLL
        datas = []
        hitacc = None
        for k in range(_UNROLL):
            start = base_off + k * 16
            valid = lane < (_EPW - start)
            a = e0_v[pl.ds(start, 16)]
            b = e1_v[pl.ds(start, 16)]
            plsc.addupdate_scatter(hist_v, [a], ones, mask=valid)
            plsc.addupdate_scatter(hist_v, [b], ones, mask=valid)
            ma = plsc.load_gather(smark_v, [a], mask=valid)
            mb = plsc.load_gather(smark_v, [b], mask=valid)
            hm = valid & ((ma | mb) != 0)
            hitacc = hm if hitacc is None else (hitacc | hm)
            datas.append((valid, a, b, ma, mb))

        @pl.when(jnp.any(hitacc))
        def _():
            for valid, a, b, ma, mb in datas:
                for s in range(_NSPEC):
                    bit = jnp.int32(1 << s)
                    m0 = valid & ((ma & bit) != 0)

                    @pl.when(jnp.any(m0))
                    def _():
                        idx = jnp.where(m0, s * _NP + b, _NSPEC * _NP + lane)
                        pltpu.sync_copy(ones_v, shared_cs.at[idx], add=True)

                    m1 = valid & ((mb & bit) != 0)

                    @pl.when(jnp.any(m1))
                    def _():
                        idx = jnp.where(m1, s * _NP + a, _NSPEC * _NP + lane)
                        pltpu.sync_copy(ones_v, shared_cs.at[idx], add=True)
        return carry

    lax.fori_loop(0, (_EPW + 16 * _UNROLL - 1) // (16 * _UNROLL), body, 0)

    pltpu.sync_copy(hist_v, out_deg_hbm.at[wid])
    plsc.subcore_barrier()

    @pl.when(sid == 0)
    def _():
        pltpu.sync_copy(shared_cs, out_cs_hbm.at[cid])


def _sc_histograms(ei, spec):
    zcs = jnp.zeros((_CSLEN,), jnp.float32)
    zi = jnp.zeros((_NP,), jnp.int32)
    mesh = plsc.VectorSubcoreMesh(core_axis_name="c", subcore_axis_name="s",
                                  num_cores=_NC, num_subcores=_NSUB)
    epad = _EPW + 16 * _UNROLL
    f = pl.kernel(
        _hist_body,
        out_type=(jax.ShapeDtypeStruct((_NW, _NP), jnp.float32),
                  jax.ShapeDtypeStruct((_NC, _CSLEN), jnp.float32)),
        mesh=mesh,
        compiler_params=pltpu.CompilerParams(needs_layout_passes=False),
        scratch_types=[
            pltpu.VMEM((epad,), jnp.int32),
            pltpu.VMEM((epad,), jnp.int32),
            pltpu.VMEM((16,), jnp.int32),
            pltpu.VMEM((_NP,), jnp.float32),
            pltpu.VMEM((_NP,), jnp.int32),
            pltpu.VMEM((16,), jnp.float32),
            pltpu.VMEM_SHARED((_CSLEN,), jnp.float32),
        ],
    )
    return f(ei, spec, zcs, zi)


_NCH = _NP // _CH


def _c0dot(w_ref, x):
    return jnp.dot(w_ref[...], x, preferred_element_type=jnp.float32)


def _dense_body(degp_ref, csp_ref, hsht_ref, relt_ref, stht_ref,
                w0t_ref, wlht_ref, wlqt_ref, wm1t_ref, wm2t_ref, w1t_ref,
                b0t_ref, b1t_ref, blint_ref, bm1t_ref, bm2_ref,
                h0_ref, t_ref, out_ref, aggsum_ref, aggmax_ref, degt_ref,
                degsum_ref, const_ref, mean_ref):
    i = pl.program_id(0)

    @pl.when(i == 0)
    def _():
        acc = degp_ref[0:1, :]
        for w in range(1, _NW):
            acc = acc + degp_ref[w:w + 1, :]
        degsum_ref[...] = acc
        mean_ref[0, 0] = jnp.sum(jnp.log(acc + 1.0)) / float(_N)

        relt = relt_ref[...]
        hsht = hsht_ref[...]
        gate0 = jax.nn.sigmoid(
            jnp.sum(stht_ref[...] * relt, axis=0, keepdims=True)
            / math.sqrt(float(_D)))
        mt = gate0 * hsht * relt
        const_ref[:, 0:_B] = _c0dot(w0t_ref, mt)
        const_ref[:, _B:2 * _B] = _c0dot(w0t_ref, jnp.maximum(mt, 0.0))
        const_ref[:, 2 * _B:3 * _B] = \
            _c0dot(wlqt_ref, relt) + blint_ref[...]
        const_ref[:, 3 * _B:4 * _B] = hsht * relt

    mean_ld = mean_ref[0, 0]
    off = i * _CH
    deg = degsum_ref[0:1, pl.ds(off, _CH)]
    scal = jnp.log(deg + 1.0) / mean_ld
    degc = jnp.maximum(deg, 1.0)
    lane_ids = lax.broadcasted_iota(jnp.int32, (1, _CH), 1) + off

    def csrow(s):
        return (csp_ref[0:1, pl.ds(s * _NP + off, _CH)]
                + csp_ref[1:2, pl.ds(s * _NP + off, _CH)])

    for bb in range(_B):
        cb = csrow(bb)
        sbt = jnp.concatenate(
            [csrow(_B + bb * _NEG + k) for k in range(_NEG)], axis=0)
        u_b = const_ref[:, bb:bb + 1]
        w_b = const_ref[:, _B + bb:_B + bb + 1]
        q_b = const_ref[:, 2 * _B + bb:2 * _B + bb + 1]
        bnd_b = const_ref[:, 3 * _B + bb:3 * _B + bb + 1]

        a_coef = scal * cb / degc \
            + jnp.where((cb > 0) & (cb == deg), scal, 0.0)
        b_coef = jnp.where((cb > 0) & (cb < deg), scal, 0.0)
        hid1 = jnp.maximum(u_b * a_coef + w_b * b_coef + b0t_ref[...], 0.0)
        hid1 = hid1 + jnp.where(lane_ids == h0_ref[bb], 1.0, 0.0) * bnd_b

        z1 = jnp.maximum(_c0dot(wlht_ref, hid1) + q_b, 0.0)
        z2 = jnp.maximum(_c0dot(wm1t_ref, z1) + bm1t_ref[...], 0.0)
        s1 = _c0dot(wm2t_ref, z2) + bm2_ref[...]
        gate1 = jax.nn.sigmoid(s1)
        g = gate1 * hid1

        part_sum = lax.dot_general(g, sbt, (((1,), (1,)), ((), ())),
                                   preferred_element_type=jnp.float32)
        neg_inf = jnp.float32(-jnp.inf)
        maxes = []
        degts = []
        for k in range(_NEG):
            gm = jnp.where(sbt[k:k + 1, :] > 0.0, g, neg_inf)
            maxes.append(jnp.max(gm, axis=1, keepdims=True))
            degts.append(jnp.sum(sbt[k:k + 1, :], axis=1, keepdims=True))
        part_max = jnp.concatenate(maxes, axis=1)
        part_degt = jnp.concatenate(degts, axis=1)

        lo, hi = bb * _NEG, (bb + 1) * _NEG

        @pl.when(i == 0)
        def _():
            aggsum_ref[:, lo:hi] = part_sum
            aggmax_ref[:, lo:hi] = part_max
            degt_ref[:, lo:hi] = part_degt

        @pl.when(i > 0)
        def _():
            aggsum_ref[:, lo:hi] = aggsum_ref[:, lo:hi] + part_sum
            aggmax_ref[:, lo:hi] = jnp.maximum(aggmax_ref[:, lo:hi], part_max)
            degt_ref[:, lo:hi] = degt_ref[:, lo:hi] + part_degt

    @pl.when(i == _NCH - 1)
    def _():
        degt = degt_ref[...]
        scal_t = jnp.log(degt + 1.0) / mean_ld
        agg2 = (aggsum_ref[...] / jnp.maximum(degt, 1.0)
                + jnp.where(degt > 0, aggmax_ref[...], 0.0)) * scal_t
        hid2 = jnp.maximum(_c0dot(w1t_ref, agg2) + b1t_ref[...], 0.0)

        bnd8 = jnp.concatenate(
            [const_ref[:, 3 * _B + bb:3 * _B + bb + 1]
             for bb in range(_B) for _ in range(_NEG)], axis=1)
        q8 = jnp.concatenate(
            [const_ref[:, 2 * _B + bb:2 * _B + bb + 1]
             for bb in range(_B) for _ in range(_NEG)], axis=1)
        tmatch = jnp.concatenate(
            [jnp.where(t_ref[bb * _NEG + k] == h0_ref[bb],
                       1.0, 0.0).reshape(1, 1)
             for bb in range(_B) for k in range(_NEG)], axis=1)
        hid2 = hid2 + tmatch * bnd8

        z1f = jnp.maximum(_c0dot(wlht_ref, hid2) + q8, 0.0)
        z2f = jnp.maximum(_c0dot(wm1t_ref, z1f) + bm1t_ref[...], 0.0)
        s2 = _c0dot(wm2t_ref, z2f) + bm2_ref[...]
        out_ref[...] = s2


def kernel(h_index, r_index, t_index, hidden_states, rel_hidden_states, x,
           edge_index, score_text_embs, all_index, rel_table, W0, b0, W1, b1,
           W_lin, b_lin, W_mlp1, b_mlp1, W_mlp2, b_mlp2):
    ei = edge_index.astype(jnp.int32).reshape(-1)
    h0 = h_index[:, 0].astype(jnp.int32)
    r0 = r_index[:, 0].astype(jnp.int32)
    t = t_index.astype(jnp.int32).reshape(-1)

    spec = jnp.concatenate(
        [h0, t, _N + jnp.arange(16 - _NSPEC, dtype=jnp.int32)])
    degp, cs_part = _sc_histograms(ei, spec)

    relt = rel_table[r0].T
    hsht = hidden_states[h0].T
    stht = score_text_embs[h0].T

    full = lambda shape: pl.BlockSpec(shape, lambda i: (0,) * len(shape))
    out = pl.pallas_call(
        _dense_body,
        grid=(_NCH,),
        in_specs=[
            full((_NW, _NP)), full((_NC, _CSLEN)),
            full((_D, _B)), full((_D, _B)), full((_D, _B)),
            full((_D, _D)), full((_D, _D)), full((_D, _D)),
            full((2 * _D, _D)), full((1, 2 * _D)), full((_D, _D)),
            full((_D, 1)), full((_D, 1)), full((_D, 1)),
            full((2 * _D, 1)), full((1, 1)),
            pl.BlockSpec(memory_space=pltpu.SMEM),
            pl.BlockSpec(memory_space=pltpu.SMEM),
        ],
        out_specs=pl.BlockSpec((1, _NT), lambda i: (0, 0)),
        out_shape=jax.ShapeDtypeStruct((1, _NT), jnp.float32),
        scratch_shapes=[
            pltpu.VMEM((_D, _NT), jnp.float32),
            pltpu.VMEM((_D, _NT), jnp.float32),
            pltpu.VMEM((1, _NT), jnp.float32),
            pltpu.VMEM((1, _NP), jnp.float32),
            pltpu.VMEM((_D, 4 * _B), jnp.float32),
            pltpu.SMEM((1, 1), jnp.float32),
        ],
    )(degp, cs_part, hsht, relt, stht,
      W0.T, W_lin[:_D].T, W_lin[_D:].T, W_mlp1.T, W_mlp2.T, W1.T,
      b0.reshape(_D, 1), b1.reshape(_D, 1), b_lin.reshape(_D, 1),
      b_mlp1.reshape(2 * _D, 1), b_mlp2.reshape(1, 1), h0, t)
    return out.reshape(_B, _NEG)


# speedup vs baseline: 1.6194x; 1.2105x over previous
import math

import jax
import jax.numpy as jnp
from jax import lax
from jax.experimental import pallas as pl
from jax.experimental.pallas import tpu as pltpu
from jax.experimental.pallas import tpu_sc as plsc

_N = 10000
_D = 128
_B = 2
_NEG = 4
_E = 160000
_CH = 2048
_NP = 10240
_NT = _B * _NEG
_NSPEC = _B + _NT
_NC = 2
_NSUB = 16
_NW = _NC * _NSUB
_EPW = _E // _NW
_DROWS = _NP // 16
_CSLEN = _NSPEC * _NP + 16


_UNROLL = 2


def _hist_body(ei_hbm, spec_hbm, zcs_hbm, zi_hbm,
               out_deg_hbm, out_cs_hbm,
               e0_v, e1_v, spec_v, hist_v, smark_v, ones_v, shared_cs):
    cid = lax.axis_index("c")
    sid = lax.axis_index("s")
    wid = sid * _NC + cid
    base = wid * _EPW

    @pl.when(sid == 0)
    def _():
        pltpu.sync_copy(zcs_hbm, shared_cs)

    pltpu.sync_copy(ei_hbm.at[pl.ds(base, _EPW)], e0_v.at[pl.ds(0, _EPW)])
    pltpu.sync_copy(ei_hbm.at[pl.ds(_E + base, _EPW)],
                    e1_v.at[pl.ds(0, _EPW)])
    pltpu.sync_copy(spec_hbm, spec_v)
    pltpu.sync_copy(zcs_hbm.at[pl.ds(0, _NP)], hist_v)
    pltpu.sync_copy(zi_hbm, smark_v)
    ones_v[...] = jnp.ones((16,), jnp.float32)

    ones = jnp.ones((16,), jnp.float32)
    lane = lax.iota(jnp.int32, 16)
    bitvals = jnp.where(lane < _NSPEC, jnp.left_shift(1, lane), 0)
    plsc.addupdate_scatter(smark_v, [spec_v[...]], bitvals)
    plsc.subcore_barrier()

    def halfbody(start):
        valid = lane < (_EPW - start)
        a = e0_v[pl.ds(start, 16)]
        b = e1_v[pl.ds(start, 16)]
        plsc.addupdate_scatter(hist_v, [a], ones, mask=valid)
        plsc.addupdate_scatter(hist_v, [b], ones, mask=valid)
        ma = plsc.load_gather(smark_v, [a], mask=valid)
        mb = plsc.load_gather(smark_v, [b], mask=valid)

        @pl.when(jnp.any(valid & ((ma | mb) != 0)))
        def _():
            for s in range(_NSPEC):
                bit = jnp.int32(1 << s)
                m0 = valid & ((ma & bit) != 0)

                @pl.when(jnp.any(m0))
                def _():
                    idx = jnp.where(m0, s * _NP + b, _NSPEC * _NP + lane)
                    pltpu.sync_copy(ones_v, shared_cs.at[idx], add=True)

                m1 = valid & ((mb & bit) != 0)

                @pl.when(jnp.any(m1))
                def _():
                    idx = jnp.where(m1, s * _NP + a, _NSPEC * _NP + lane)
                    pltpu.sync_copy(ones_v, shared_cs.at[idx], add=True)

    def body(j, carry):
        for k in range(_UNROLL):
            halfbody(j * 16 * _UNROLL + k * 16)
        return carry

    lax.fori_loop(0, (_EPW + 16 * _UNROLL - 1) // (16 * _UNROLL), body, 0)

    pltpu.sync_copy(hist_v, out_deg_hbm.at[wid])
    plsc.subcore_barrier()

    @pl.when(sid == 0)
    def _():
        pltpu.sync_copy(shared_cs, out_cs_hbm.at[cid])


def _sc_histograms(ei, spec):
    zcs = jnp.zeros((_CSLEN,), jnp.float32)
    zi = jnp.zeros((_NP,), jnp.int32)
    mesh = plsc.VectorSubcoreMesh(core_axis_name="c", subcore_axis_name="s",
                                  num_cores=_NC, num_subcores=_NSUB)
    epad = _EPW + 16 * _UNROLL
    f = pl.kernel(
        _hist_body,
        out_type=(jax.ShapeDtypeStruct((_NW, _NP), jnp.float32),
                  jax.ShapeDtypeStruct((_NC, _CSLEN), jnp.float32)),
        mesh=mesh,
        compiler_params=pltpu.CompilerParams(needs_layout_passes=False),
        scratch_types=[
            pltpu.VMEM((epad,), jnp.int32),
            pltpu.VMEM((epad,), jnp.int32),
            pltpu.VMEM((16,), jnp.int32),
            pltpu.VMEM((_NP,), jnp.float32),
            pltpu.VMEM((_NP,), jnp.int32),
            pltpu.VMEM((16,), jnp.float32),
            pltpu.VMEM_SHARED((_CSLEN,), jnp.float32),
        ],
    )
    return f(ei, spec, zcs, zi)


_NCH = _NP // _CH


def _c0dot(w_ref, x):
    return jnp.dot(w_ref[...], x, preferred_element_type=jnp.float32)


def _dense_body(degp_ref, csp_ref, hsht_ref, relt_ref, stht_ref,
                w0t_ref, wlht_ref, wlqt_ref, wm1t_ref, wm2t_ref, w1t_ref,
                b0t_ref, b1t_ref, blint_ref, bm1t_ref, bm2_ref,
                h0_ref, t_ref, out_ref, aggsum_ref, aggmax_ref, degt_ref,
                degsum_ref, const_ref, mean_ref):
    i = pl.program_id(0)

    @pl.when(i == 0)
    def _():
        acc = degp_ref[0:1, :]
        for w in range(1, _NW):
            acc = acc + degp_ref[w:w + 1, :]
        degsum_ref[...] = acc
        mean_ref[0, 0] = jnp.sum(jnp.log(acc + 1.0)) / float(_N)

        relt = relt_ref[...]
        hsht = hsht_ref[...]
        gate0 = jax.nn.sigmoid(
            jnp.sum(stht_ref[...] * relt, axis=0, keepdims=True)
            / math.sqrt(float(_D)))
        mt = gate0 * hsht * relt
        const_ref[:, 0:_B] = _c0dot(w0t_ref, mt)
        const_ref[:, _B:2 * _B] = _c0dot(w0t_ref, jnp.maximum(mt, 0.0))
        const_ref[:, 2 * _B:3 * _B] = \
            _c0dot(wlqt_ref, relt) + blint_ref[...]
        const_ref[:, 3 * _B:4 * _B] = hsht * relt

    mean_ld = mean_ref[0, 0]
    off = i * _CH
    deg = degsum_ref[0:1, pl.ds(off, _CH)]
    scal = jnp.log(deg + 1.0) / mean_ld
    degc = jnp.maximum(deg, 1.0)
    lane_ids = lax.broadcasted_iota(jnp.int32, (1, _CH), 1) + off

    def csrow(s):
        return (csp_ref[0:1, pl.ds(s * _NP + off, _CH)]
                + csp_ref[1:2, pl.ds(s * _NP + off, _CH)])

    for bb in range(_B):
        cb = csrow(bb)
        sbt = jnp.concatenate(
            [csrow(_B + bb * _NEG + k) for k in range(_NEG)], axis=0)
        u_b = const_ref[:, bb:bb + 1]
        w_b = const_ref[:, _B + bb:_B + bb + 1]
        q_b = const_ref[:, 2 * _B + bb:2 * _B + bb + 1]
        bnd_b = const_ref[:, 3 * _B + bb:3 * _B + bb + 1]

        a_coef = scal * cb / degc \
            + jnp.where((cb > 0) & (cb == deg), scal, 0.0)
        b_coef = jnp.where((cb > 0) & (cb < deg), scal, 0.0)
        hid1 = jnp.maximum(u_b * a_coef + w_b * b_coef + b0t_ref[...], 0.0)
        hid1 = hid1 + jnp.where(lane_ids == h0_ref[bb], 1.0, 0.0) * bnd_b

        z1 = jnp.maximum(_c0dot(wlht_ref, hid1) + q_b, 0.0)
        z2 = jnp.maximum(_c0dot(wm1t_ref, z1) + bm1t_ref[...], 0.0)
        s1 = _c0dot(wm2t_ref, z2) + bm2_ref[...]
        gate1 = jax.nn.sigmoid(s1)
        g = gate1 * hid1

        part_sum = lax.dot_general(g, sbt, (((1,), (1,)), ((), ())),
                                   preferred_element_type=jnp.float32)
        neg_inf = jnp.float32(-jnp.inf)
        maxes = []
        degts = []
        for k in range(_NEG):
            gm = jnp.where(sbt[k:k + 1, :] > 0.0, g, neg_inf)
            maxes.append(jnp.max(gm, axis=1, keepdims=True))
            degts.append(jnp.sum(sbt[k:k + 1, :], axis=1, keepdims=True))
        part_max = jnp.concatenate(maxes, axis=1)
        part_degt = jnp.concatenate(degts, axis=1)

        lo, hi = bb * _NEG, (bb + 1) * _NEG

        @pl.when(i == 0)
        def _():
            aggsum_ref[:, lo:hi] = part_sum
            aggmax_ref[:, lo:hi] = part_max
            degt_ref[:, lo:hi] = part_degt

        @pl.when(i > 0)
        def _():
            aggsum_ref[:, lo:hi] = aggsum_ref[:, lo:hi] + part_sum
            aggmax_ref[:, lo:hi] = jnp.maximum(aggmax_ref[:, lo:hi], part_max)
            degt_ref[:, lo:hi] = degt_ref[:, lo:hi] + part_degt

    @pl.when(i == _NCH - 1)
    def _():
        degt = degt_ref[...]
        scal_t = jnp.log(degt + 1.0) / mean_ld
        agg2 = (aggsum_ref[...] / jnp.maximum(degt, 1.0)
                + jnp.where(degt > 0, aggmax_ref[...], 0.0)) * scal_t
        hid2 = jnp.maximum(_c0dot(w1t_ref, agg2) + b1t_ref[...], 0.0)

        bnd8 = jnp.concatenate(
            [const_ref[:, 3 * _B + bb:3 * _B + bb + 1]
             for bb in range(_B) for _ in range(_NEG)], axis=1)
        q8 = jnp.concatenate(
            [const_ref[:, 2 * _B + bb:2 * _B + bb + 1]
             for bb in range(_B) for _ in range(_NEG)], axis=1)
        tmatch = jnp.concatenate(
            [jnp.where(t_ref[bb * _NEG + k] == h0_ref[bb],
                       1.0, 0.0).reshape(1, 1)
             for bb in range(_B) for k in range(_NEG)], axis=1)
        hid2 = hid2 + tmatch * bnd8

        z1f = jnp.maximum(_c0dot(wlht_ref, hid2) + q8, 0.0)
        z2f = jnp.maximum(_c0dot(wm1t_ref, z1f) + bm1t_ref[...], 0.0)
        s2 = _c0dot(wm2t_ref, z2f) + bm2_ref[...]
        out_ref[...] = s2


def kernel(h_index, r_index, t_index, hidden_states, rel_hidden_states, x,
           edge_index, score_text_embs, all_index, rel_table, W0, b0, W1, b1,
           W_lin, b_lin, W_mlp1, b_mlp1, W_mlp2, b_mlp2):
    ei = edge_index.astype(jnp.int32).reshape(-1)
    h0 = h_index[:, 0].astype(jnp.int32)
    r0 = r_index[:, 0].astype(jnp.int32)
    t = t_index.astype(jnp.int32).reshape(-1)

    spec = jnp.concatenate(
        [h0, t, _N + jnp.arange(16 - _NSPEC, dtype=jnp.int32)])
    degp, cs_part = _sc_histograms(ei, spec)

    relt = rel_table[r0].T
    hsht = hidden_states[h0].T
    stht = score_text_embs[h0].T

    full = lambda shape: pl.BlockSpec(shape, lambda i: (0,) * len(shape))
    out = pl.pallas_call(
        _dense_body,
        grid=(_NCH,),
        in_specs=[
            full((_NW, _NP)), full((_NC, _CSLEN)),
            full((_D, _B)), full((_D, _B)), full((_D, _B)),
            full((_D, _D)), full((_D, _D)), full((_D, _D)),
            full((2 * _D, _D)), full((1, 2 * _D)), full((_D, _D)),
            full((_D, 1)), full((_D, 1)), full((_D, 1)),
            full((2 * _D, 1)), full((1, 1)),
            pl.BlockSpec(memory_space=pltpu.SMEM),
            pl.BlockSpec(memory_space=pltpu.SMEM),
        ],
        out_specs=pl.BlockSpec((1, _NT), lambda i: (0, 0)),
        out_shape=jax.ShapeDtypeStruct((1, _NT), jnp.float32),
        scratch_shapes=[
            pltpu.VMEM((_D, _NT), jnp.float32),
            pltpu.VMEM((_D, _NT), jnp.float32),
            pltpu.VMEM((1, _NT), jnp.float32),
            pltpu.VMEM((1, _NP), jnp.float32),
            pltpu.VMEM((_D, 4 * _B), jnp.float32),
            pltpu.SMEM((1, 1), jnp.float32),
        ],
    )(degp, cs_part, hsht, relt, stht,
      W0.T, W_lin[:_D].T, W_lin[_D:].T, W_mlp1.T, W_mlp2.T, W1.T,
      b0.reshape(_D, 1), b1.reshape(_D, 1), b_lin.reshape(_D, 1),
      b_mlp1.reshape(2 * _D, 1), b_mlp2.reshape(1, 1), h0, t)
    return out.reshape(_B, _NEG)


# speedup vs baseline: 1.6691x; 1.0307x over previous
import math

import jax
import jax.numpy as jnp
from jax import lax
from jax.experimental import pallas as pl
from jax.experimental.pallas import tpu as pltpu
from jax.experimental.pallas import tpu_sc as plsc

_N = 10000
_D = 128
_B = 2
_NEG = 4
_E = 160000
_CH = 10240
_NP = 10240
_NT = _B * _NEG
_NSPEC = _B + _NT
_NC = 2
_NSUB = 16
_NW = _NC * _NSUB
_EPW = _E // _NW
_DROWS = _NP // 16
_CSLEN = _NSPEC * _NP + 16


_UNROLL = 2


def _hist_body(ei_hbm, spec_hbm, zcs_hbm, zi_hbm,
               out_deg_hbm, out_cs_hbm,
               e0_v, e1_v, spec_v, hist_v, smark_v, ones_v, shared_cs):
    cid = lax.axis_index("c")
    sid = lax.axis_index("s")
    wid = sid * _NC + cid
    base = wid * _EPW

    @pl.when(sid == 0)
    def _():
        pltpu.sync_copy(zcs_hbm, shared_cs)

    pltpu.sync_copy(ei_hbm.at[pl.ds(base, _EPW)], e0_v.at[pl.ds(0, _EPW)])
    pltpu.sync_copy(ei_hbm.at[pl.ds(_E + base, _EPW)],
                    e1_v.at[pl.ds(0, _EPW)])
    pltpu.sync_copy(spec_hbm, spec_v)
    pltpu.sync_copy(zcs_hbm.at[pl.ds(0, _NP)], hist_v)
    pltpu.sync_copy(zi_hbm, smark_v)
    ones_v[...] = jnp.ones((16,), jnp.float32)

    ones = jnp.ones((16,), jnp.float32)
    lane = lax.iota(jnp.int32, 16)
    bitvals = jnp.where(lane < _NSPEC, jnp.left_shift(1, lane), 0)
    plsc.addupdate_scatter(smark_v, [spec_v[...]], bitvals)
    plsc.subcore_barrier()

    def halfbody(start):
        valid = lane < (_EPW - start)
        a = e0_v[pl.ds(start, 16)]
        b = e1_v[pl.ds(start, 16)]
        plsc.addupdate_scatter(hist_v, [a], ones, mask=valid)
        plsc.addupdate_scatter(hist_v, [b], ones, mask=valid)
        ma = plsc.load_gather(smark_v, [a], mask=valid)
        mb = plsc.load_gather(smark_v, [b], mask=valid)

        @pl.when(jnp.any(valid & ((ma | mb) != 0)))
        def _():
            for s in range(_NSPEC):
                bit = jnp.int32(1 << s)
                m0 = valid & ((ma & bit) != 0)

                @pl.when(jnp.any(m0))
                def _():
                    idx = jnp.where(m0, s * _NP + b, _NSPEC * _NP + lane)
                    pltpu.sync_copy(ones_v, shared_cs.at[idx], add=True)

                m1 = valid & ((mb & bit) != 0)

                @pl.when(jnp.any(m1))
                def _():
                    idx = jnp.where(m1, s * _NP + a, _NSPEC * _NP + lane)
                    pltpu.sync_copy(ones_v, shared_cs.at[idx], add=True)

    def body(j, carry):
        for k in range(_UNROLL):
            halfbody(j * 16 * _UNROLL + k * 16)
        return carry

    lax.fori_loop(0, (_EPW + 16 * _UNROLL - 1) // (16 * _UNROLL), body, 0)

    pltpu.sync_copy(hist_v, out_deg_hbm.at[wid])
    plsc.subcore_barrier()

    @pl.when(sid == 0)
    def _():
        pltpu.sync_copy(shared_cs, out_cs_hbm.at[cid])


def _sc_histograms(ei, spec):
    zcs = jnp.zeros((_CSLEN,), jnp.float32)
    zi = jnp.zeros((_NP,), jnp.int32)
    mesh = plsc.VectorSubcoreMesh(core_axis_name="c", subcore_axis_name="s",
                                  num_cores=_NC, num_subcores=_NSUB)
    epad = _EPW + 16 * _UNROLL
    f = pl.kernel(
        _hist_body,
        out_type=(jax.ShapeDtypeStruct((_NW, _NP), jnp.float32),
                  jax.ShapeDtypeStruct((_NC, _CSLEN), jnp.float32)),
        mesh=mesh,
        compiler_params=pltpu.CompilerParams(needs_layout_passes=False),
        scratch_types=[
            pltpu.VMEM((epad,), jnp.int32),
            pltpu.VMEM((epad,), jnp.int32),
            pltpu.VMEM((16,), jnp.int32),
            pltpu.VMEM((_NP,), jnp.float32),
            pltpu.VMEM((_NP,), jnp.int32),
            pltpu.VMEM((16,), jnp.float32),
            pltpu.VMEM_SHARED((_CSLEN,), jnp.float32),
        ],
    )
    return f(ei, spec, zcs, zi)


_NCH = _NP // _CH


def _c0dot(w_ref, x):
    return jnp.dot(w_ref[...], x, preferred_element_type=jnp.float32)


def _dense_body(degp_ref, csp_ref, hsht_ref, relt_ref, stht_ref,
                w0t_ref, wlht_ref, wlqt_ref, wm1t_ref, wm2t_ref, w1t_ref,
                b0t_ref, b1t_ref, blint_ref, bm1t_ref, bm2_ref,
                h0_ref, t_ref, out_ref, aggsum_ref, aggmax_ref, degt_ref,
                degsum_ref, const_ref, mean_ref):
    i = pl.program_id(0)

    @pl.when(i == 0)
    def _():
        acc = degp_ref[0:1, :]
        for w in range(1, _NW):
            acc = acc + degp_ref[w:w + 1, :]
        degsum_ref[...] = acc
        mean_ref[0, 0] = jnp.sum(jnp.log(acc + 1.0)) / float(_N)

        relt = relt_ref[...]
        hsht = hsht_ref[...]
        gate0 = jax.nn.sigmoid(
            jnp.sum(stht_ref[...] * relt, axis=0, keepdims=True)
            / math.sqrt(float(_D)))
        mt = gate0 * hsht * relt
        const_ref[:, 0:_B] = _c0dot(w0t_ref, mt)
        const_ref[:, _B:2 * _B] = _c0dot(w0t_ref, jnp.maximum(mt, 0.0))
        const_ref[:, 2 * _B:3 * _B] = \
            _c0dot(wlqt_ref, relt) + blint_ref[...]
        const_ref[:, 3 * _B:4 * _B] = hsht * relt

    mean_ld = mean_ref[0, 0]
    off = i * _CH
    deg = degsum_ref[0:1, pl.ds(off, _CH)]
    scal = jnp.log(deg + 1.0) / mean_ld
    degc = jnp.maximum(deg, 1.0)
    lane_ids = lax.broadcasted_iota(jnp.int32, (1, _CH), 1) + off

    def csrow(s):
        return (csp_ref[0:1, pl.ds(s * _NP + off, _CH)]
                + csp_ref[1:2, pl.ds(s * _NP + off, _CH)])

    for bb in range(_B):
        cb = csrow(bb)
        sbt = jnp.concatenate(
            [csrow(_B + bb * _NEG + k) for k in range(_NEG)], axis=0)
        u_b = const_ref[:, bb:bb + 1]
        w_b = const_ref[:, _B + bb:_B + bb + 1]
        q_b = const_ref[:, 2 * _B + bb:2 * _B + bb + 1]
        bnd_b = const_ref[:, 3 * _B + bb:3 * _B + bb + 1]

        a_coef = scal * cb / degc \
            + jnp.where((cb > 0) & (cb == deg), scal, 0.0)
        b_coef = jnp.where((cb > 0) & (cb < deg), scal, 0.0)
        hid1 = jnp.maximum(u_b * a_coef + w_b * b_coef + b0t_ref[...], 0.0)
        hid1 = hid1 + jnp.where(lane_ids == h0_ref[bb], 1.0, 0.0) * bnd_b

        z1 = jnp.maximum(_c0dot(wlht_ref, hid1) + q_b, 0.0)
        z2 = jnp.maximum(_c0dot(wm1t_ref, z1) + bm1t_ref[...], 0.0)
        s1 = _c0dot(wm2t_ref, z2) + bm2_ref[...]
        gate1 = jax.nn.sigmoid(s1)
        g = gate1 * hid1

        part_sum = lax.dot_general(g, sbt, (((1,), (1,)), ((), ())),
                                   preferred_element_type=jnp.float32)
        neg_inf = jnp.float32(-jnp.inf)
        maxes = []
        degts = []
        for k in range(_NEG):
            gm = jnp.where(sbt[k:k + 1, :] > 0.0, g, neg_inf)
            maxes.append(jnp.max(gm, axis=1, keepdims=True))
            degts.append(jnp.sum(sbt[k:k + 1, :], axis=1, keepdims=True))
        part_max = jnp.concatenate(maxes, axis=1)
        part_degt = jnp.concatenate(degts, axis=1)

        lo, hi = bb * _NEG, (bb + 1) * _NEG

        @pl.when(i == 0)
        def _():
            aggsum_ref[:, lo:hi] = part_sum
            aggmax_ref[:, lo:hi] = part_max
            degt_ref[:, lo:hi] = part_degt

        @pl.when(i > 0)
        def _():
            aggsum_ref[:, lo:hi] = aggsum_ref[:, lo:hi] + part_sum
            aggmax_ref[:, lo:hi] = jnp.maximum(aggmax_ref[:, lo:hi], part_max)
            degt_ref[:, lo:hi] = degt_ref[:, lo:hi] + part_degt

    @pl.when(i == _NCH - 1)
    def _():
        degt = degt_ref[...]
        scal_t = jnp.log(degt + 1.0) / mean_ld
        agg2 = (aggsum_ref[...] / jnp.maximum(degt, 1.0)
                + jnp.where(degt > 0, aggmax_ref[...], 0.0)) * scal_t
        hid2 = jnp.maximum(_c0dot(w1t_ref, agg2) + b1t_ref[...], 0.0)

        bnd8 = jnp.concatenate(
            [const_ref[:, 3 * _B + bb:3 * _B + bb + 1]
             for bb in range(_B) for _ in range(_NEG)], axis=1)
        q8 = jnp.concatenate(
            [const_ref[:, 2 * _B + bb:2 * _B + bb + 1]
             for bb in range(_B) for _ in range(_NEG)], axis=1)
        tmatch = jnp.concatenate(
            [jnp.where(t_ref[bb * _NEG + k] == h0_ref[bb],
                       1.0, 0.0).reshape(1, 1)
             for bb in range(_B) for k in range(_NEG)], axis=1)
        hid2 = hid2 + tmatch * bnd8

        z1f = jnp.maximum(_c0dot(wlht_ref, hid2) + q8, 0.0)
        z2f = jnp.maximum(_c0dot(wm1t_ref, z1f) + bm1t_ref[...], 0.0)
        s2 = _c0dot(wm2t_ref, z2f) + bm2_ref[...]
        out_ref[...] = s2


def kernel(h_index, r_index, t_index, hidden_states, rel_hidden_states, x,
           edge_index, score_text_embs, all_index, rel_table, W0, b0, W1, b1,
           W_lin, b_lin, W_mlp1, b_mlp1, W_mlp2, b_mlp2):
    ei = edge_index.astype(jnp.int32).reshape(-1)
    h0 = h_index[:, 0].astype(jnp.int32)
    r0 = r_index[:, 0].astype(jnp.int32)
    t = t_index.astype(jnp.int32).reshape(-1)

    spec = jnp.concatenate(
        [h0, t, _N + jnp.arange(16 - _NSPEC, dtype=jnp.int32)])
    degp, cs_part = _sc_histograms(ei, spec)

    relt = rel_table[r0].T
    hsht = hidden_states[h0].T
    stht = score_text_embs[h0].T

    full = lambda shape: pl.BlockSpec(shape, lambda i: (0,) * len(shape))
    out = pl.pallas_call(
        _dense_body,
        grid=(_NCH,),
        in_specs=[
            full((_NW, _NP)), full((_NC, _CSLEN)),
            full((_D, _B)), full((_D, _B)), full((_D, _B)),
            full((_D, _D)), full((_D, _D)), full((_D, _D)),
            full((2 * _D, _D)), full((1, 2 * _D)), full((_D, _D)),
            full((_D, 1)), full((_D, 1)), full((_D, 1)),
            full((2 * _D, 1)), full((1, 1)),
            pl.BlockSpec(memory_space=pltpu.SMEM),
            pl.BlockSpec(memory_space=pltpu.SMEM),
        ],
        out_specs=pl.BlockSpec((1, _NT), lambda i: (0, 0)),
        out_shape=jax.ShapeDtypeStruct((1, _NT), jnp.float32),
        scratch_shapes=[
            pltpu.VMEM((_D, _NT), jnp.float32),
            pltpu.VMEM((_D, _NT), jnp.float32),
            pltpu.VMEM((1, _NT), jnp.float32),
            pltpu.VMEM((1, _NP), jnp.float32),
            pltpu.VMEM((_D, 4 * _B), jnp.float32),
            pltpu.SMEM((1, 1), jnp.float32),
        ],
    )(degp, cs_part, hsht, relt, stht,
      W0.T, W_lin[:_D].T, W_lin[_D:].T, W_mlp1.T, W_mlp2.T, W1.T,
      b0.reshape(_D, 1), b1.reshape(_D, 1), b_lin.reshape(_D, 1),
      b_mlp1.reshape(2 * _D, 1), b_mlp2.reshape(1, 1), h0, t)
    return out.reshape(_B, _NEG)
